# baseline (speedup 1.0000x reference)
import functools

import jax
import jax.numpy as jnp
from jax import lax
from jax.experimental import pallas as pl
from jax.experimental.pallas import tpu as pltpu
from jax.experimental.pallas import tpu_sc as plsc

_VOCAB = 1000000
_D = 64
_B = 4096
_L = 60
_Q = 50

_NC = 2
_NS = 16
_NW = _NC * _NS
_TOTAL = _B * _Q
_PER_W = _TOTAL // _NW
_CH = 128
_NCH = _PER_W // _CH
_NBUF = 5
_NG = _NCH // _NBUF
_ROWS_W = _B // _NW
_TPW = _ROWS_W * _L
_MASK_W = _ROWS_W * _Q

_mesh = plsc.VectorSubcoreMesh(
    core_axis_name="c", subcore_axis_name="s", num_cores=_NC, num_subcores=_NS
)


def _emb_body(q_hbm, types_hbm, table_hbm, out_hbm, mask_hbm,
              idx_v, types_v, mask_v, *scratch):
  bufs_a = scratch[:_NBUF]
  bufs_b = scratch[_NBUF:2 * _NBUF]
  sems_a = scratch[2 * _NBUF:3 * _NBUF]
  sems_b = scratch[3 * _NBUF:4 * _NBUF]
  wid = lax.axis_index("s") * _NC + lax.axis_index("c")

  pltpu.sync_copy(q_hbm.at[wid], idx_v)

  def fire_group(g, bufs, sems):
    for b in range(_NBUF):
      c = g * _NBUF + b
      pltpu.async_copy(table_hbm.at[idx_v.at[c]], bufs[b], sems[b])

  def drain_group(g, bufs, sems):
    for b in range(_NBUF):
      c = g * _NBUF + b
      pltpu.make_async_copy(table_hbm.at[idx_v.at[c]], bufs[b], sems[b]).wait()
      pltpu.sync_copy(bufs[b], out_hbm.at[pl.ds(wid * _PER_W + c * _CH, _CH)])

  fire_group(0, bufs_a, sems_a)
  fire_group(1, bufs_b, sems_b)

  pltpu.sync_copy(
      types_hbm.at[pl.ds(wid * _TPW, _TPW)], types_v.at[pl.ds(0, _TPW)])
  v_two = jnp.full((16,), 2, jnp.int32)
  v_one = jnp.full((16,), 1.0, jnp.float32)
  v_zero = jnp.full((16,), 0.0, jnp.float32)

  def mask_step(r, carry):
    for j in range(4):
      t = types_v[pl.ds(r * _L + (_L - _Q) + 16 * j, 16)]
      mask_v[pl.ds(r * _Q + 16 * j, 16)] = jnp.where(t == v_two, v_one, v_zero)
    return carry

  lax.fori_loop(0, _ROWS_W, mask_step, 0)
  pltpu.sync_copy(
      mask_v.at[pl.ds(0, _MASK_W)], mask_hbm.at[pl.ds(wid * _MASK_W, _MASK_W)])

  def pipe_step(p, carry):
    drain_group(2 * p, bufs_a, sems_a)
    fire_group(2 * p + 2, bufs_a, sems_a)
    drain_group(2 * p + 1, bufs_b, sems_b)
    fire_group(2 * p + 3, bufs_b, sems_b)
    return carry

  lax.fori_loop(0, _NG // 2 - 1, pipe_step, 0)
  drain_group(_NG - 2, bufs_a, sems_a)
  drain_group(_NG - 1, bufs_b, sems_b)


_emb_call = functools.partial(
    pl.kernel,
    out_type=[
        jax.ShapeDtypeStruct((_TOTAL, _D), jnp.float32),
        jax.ShapeDtypeStruct((_TOTAL,), jnp.float32),
    ],
    mesh=_mesh,
    compiler_params=pltpu.CompilerParams(use_tc_tiling_on_sc=False),
    scratch_types=(
        [
            pltpu.VMEM((_NCH, _CH), jnp.int32),
            pltpu.VMEM((_TPW + 16,), jnp.int32),
            pltpu.VMEM((_MASK_W + 16,), jnp.float32),
        ]
        + [pltpu.VMEM((_CH, _D), jnp.float32) for _ in range(2 * _NBUF)]
        + [pltpu.SemaphoreType.DMA for _ in range(2 * _NBUF)]
    ),
)(_emb_body)


def kernel(positions, types, object_positions, object_colors, object_shapes,
           object_materials, object_sizes, question, question_embeddings):
  q3 = question.reshape(_NW, _NCH, _CH)
  types_flat = types.reshape(-1)
  out, mask = _emb_call(q3, types_flat, question_embeddings)
  return out.reshape(_B, _Q, _D), mask.reshape(_B, 1, 1, _Q)


# speedup vs baseline: 1.2362x; 1.2362x over previous
import functools

import jax
import jax.numpy as jnp
from jax import lax
from jax.experimental import pallas as pl
from jax.experimental.pallas import tpu as pltpu
from jax.experimental.pallas import tpu_sc as plsc

_VOCAB = 1000000
_D = 64
_B = 4096
_L = 60
_Q = 50

_NC = 2
_NS = 16
_NW = _NC * _NS
_TOTAL = _B * _Q
_PPT = _TOTAL // _NS
_CPC = _D // _NC
_ROWS_W = _B // _NW
_TPW = _ROWS_W * _L
_MASK_W = _ROWS_W * _Q

_mesh = plsc.VectorSubcoreMesh(
    core_axis_name="c", subcore_axis_name="s", num_cores=_NC, num_subcores=_NS
)


def _emb_body(tab_hbm, idx_hbm, types_hbm, out_hbm, mask_hbm,
              idx_v, types_v, mask_v, buf, spm, sem_g, sem_s):
  sid = lax.axis_index("s")
  cid = lax.axis_index("c")
  wid = sid * _NC + cid
  base = cid * _CPC

  pltpu.sync_copy(idx_hbm.at[sid], idx_v)

  @pl.when(sid == 0)
  def _():
    pltpu.async_copy(tab_hbm.at[base], spm, sem_s)

  pltpu.sync_copy(
      types_hbm.at[pl.ds(wid * _TPW, _TPW)], types_v.at[pl.ds(0, _TPW)])
  v_two = jnp.full((16,), 2, jnp.int32)
  v_one = jnp.full((16,), 1.0, jnp.float32)
  v_zero = jnp.full((16,), 0.0, jnp.float32)

  def mask_step(r, carry):
    for j in range(4):
      t = types_v[pl.ds(r * _L + (_L - _Q) + 16 * j, 16)]
      mask_v[pl.ds(r * _Q + 16 * j, 16)] = jnp.where(t == v_two, v_one, v_zero)
    return carry

  lax.fori_loop(0, _ROWS_W, mask_step, 0)
  pltpu.sync_copy(
      mask_v.at[pl.ds(0, _MASK_W)], mask_hbm.at[pl.ds(wid * _MASK_W, _MASK_W)])

  def col_step(c, carry):
    row = base + c

    @pl.when(sid == 0)
    def _():
      pltpu.make_async_copy(tab_hbm.at[row], spm, sem_s).wait()

    plsc.subcore_barrier()
    pltpu.async_copy(spm.at[idx_v], buf, sem_g).wait()
    plsc.subcore_barrier()

    @pl.when((sid == 0) & (c < _CPC - 1))
    def _():
      pltpu.async_copy(tab_hbm.at[row + 1], spm, sem_s)

    pltpu.sync_copy(buf, out_hbm.at[row].at[pl.ds(sid * _PPT, _PPT)])
    return carry

  lax.fori_loop(0, _CPC, col_step, 0)


_emb_call = functools.partial(
    pl.kernel,
    out_type=[
        jax.ShapeDtypeStruct((_D, _TOTAL), jnp.float32),
        jax.ShapeDtypeStruct((_TOTAL,), jnp.float32),
    ],
    mesh=_mesh,
    scratch_types=[
        pltpu.VMEM((_PPT,), jnp.int32),
        pltpu.VMEM((_TPW + 16,), jnp.int32),
        pltpu.VMEM((_MASK_W + 16,), jnp.float32),
        pltpu.VMEM((_PPT,), jnp.float32),
        pltpu.VMEM_SHARED((_VOCAB,), jnp.float32),
        pltpu.SemaphoreType.DMA,
        pltpu.SemaphoreType.DMA,
    ],
)(_emb_body)


def kernel(positions, types, object_positions, object_colors, object_shapes,
           object_materials, object_sizes, question, question_embeddings):
  tab_t = question_embeddings.T
  idx2 = question.reshape(_NS, _PPT)
  types_flat = types.reshape(-1)
  out, mask = _emb_call(tab_t, idx2, types_flat)
  return (
      out.T.reshape(_B, _Q, _D),
      mask.reshape(_B, 1, 1, _Q),
  )


# speedup vs baseline: 1.7620x; 1.4253x over previous
import functools

import jax
import jax.numpy as jnp
from jax import lax
from jax.experimental import pallas as pl
from jax.experimental.pallas import tpu as pltpu
from jax.experimental.pallas import tpu_sc as plsc

_VOCAB = 1000000
_D = 64
_B = 4096
_L = 60
_Q = 50

_NC = 2
_NS = 16
_NW = _NC * _NS
_TOTAL = _B * _Q
_PPT = _TOTAL // _NS
_CPC = _D // _NC
_ROWS_W = _B // _NW
_TPW = _ROWS_W * _L
_MASK_W = _ROWS_W * _Q

_mesh = plsc.VectorSubcoreMesh(
    core_axis_name="c", subcore_axis_name="s", num_cores=_NC, num_subcores=_NS
)


def _emb_body(tab_hbm, idx_hbm, types_hbm, out_hbm, mask_hbm,
              idx_v, types_v, mask_v, buf, spm, sem_g, sem_s):
  sid = lax.axis_index("s")
  cid = lax.axis_index("c")
  wid = sid * _NC + cid
  base = cid * _CPC

  pltpu.sync_copy(idx_hbm.at[sid], idx_v)

  @pl.when(sid == 0)
  def _():
    pltpu.async_copy(tab_hbm.at[base], spm, sem_s)

  pltpu.sync_copy(
      types_hbm.at[pl.ds((_L - _Q) * _B + wid * _MASK_W, _MASK_W)], types_v)
  v_two = jnp.full((16,), 2, jnp.int32)
  v_one = jnp.full((16,), 1.0, jnp.float32)
  v_zero = jnp.full((16,), 0.0, jnp.float32)

  def mask_step(i, carry):
    for j in range(4):
      m0 = (i * 4 + j) * 16
      t = types_v[pl.ds(m0, 16)]
      mask_v[pl.ds(m0, 16)] = jnp.where(t == v_two, v_one, v_zero)
    return carry

  lax.fori_loop(0, _MASK_W // 64, mask_step, 0)
  pltpu.sync_copy(mask_v, mask_hbm.at[pl.ds(wid * _MASK_W, _MASK_W)])

  def col_step(c, carry):
    row = base + c

    @pl.when(sid == 0)
    def _():
      pltpu.make_async_copy(tab_hbm.at[row], spm, sem_s).wait()

    plsc.subcore_barrier()
    pltpu.async_copy(spm.at[idx_v], buf, sem_g).wait()
    plsc.subcore_barrier()

    @pl.when((sid == 0) & (c < _CPC - 1))
    def _():
      pltpu.async_copy(tab_hbm.at[row + 1], spm, sem_s)

    pltpu.sync_copy(buf, out_hbm.at[row].at[pl.ds(sid * _PPT, _PPT)])
    return carry

  lax.fori_loop(0, _CPC, col_step, 0)


_emb_call = functools.partial(
    pl.kernel,
    out_type=[
        jax.ShapeDtypeStruct((_D, _TOTAL), jnp.float32),
        jax.ShapeDtypeStruct((_TOTAL,), jnp.float32),
    ],
    mesh=_mesh,
    scratch_types=[
        pltpu.VMEM((_PPT,), jnp.int32),
        pltpu.VMEM((_MASK_W,), jnp.int32),
        pltpu.VMEM((_MASK_W,), jnp.float32),
        pltpu.VMEM((_PPT,), jnp.float32),
        pltpu.VMEM_SHARED((_VOCAB,), jnp.float32),
        pltpu.SemaphoreType.DMA,
        pltpu.SemaphoreType.DMA,
    ],
)(_emb_body)


def kernel(positions, types, object_positions, object_colors, object_shapes,
           object_materials, object_sizes, question, question_embeddings):
  tab_t = question_embeddings.T
  idx2 = question.T.reshape(_NS, _PPT)
  types_flat = types.T.reshape(-1)
  out, mask = _emb_call(tab_t, idx2, types_flat)
  out3 = out.reshape(_D, _Q, _B)
  return (
      out3.transpose(2, 1, 0),
      mask.reshape(_Q, _B).T.reshape(_B, 1, 1, _Q),
  )
